# Initial kernel scaffold; baseline (speedup 1.0000x reference)
import jax
import jax.numpy as jnp
from jax.experimental import pallas as pl


def kernel(x, edge_index, W1, b1, W2, b2):
    raise NotImplementedError("write your pallas kernel here")


# speedup vs baseline: 20.6969x; 20.6969x over previous
import functools

import jax
import jax.numpy as jnp
from jax import lax
from jax.experimental import pallas as pl
from jax.experimental.pallas import tpu as pltpu
from jax.experimental.pallas import tpu_sc as plsc

_SC_PARAMS = pltpu.CompilerParams(use_tc_tiling_on_sc=False)

NC = 2
NS = 16
NW = NC * NS
D = 16
C = 128


def _pad_rows(n):
    per_tile = -(-(n + 8) // (NS * 8)) * 8
    return per_tile * NS


def _sc_scatter_add(g, src, dst, n_pad, chunks_per_w):
    mesh = plsc.VectorSubcoreMesh(core_axis_name="c", subcore_axis_name="s")
    rows_per_tile = n_pad // NS

    @functools.partial(
        pl.kernel,
        out_type=jax.ShapeDtypeStruct((NC, n_pad, D), jnp.float32),
        mesh=mesh,
        scratch_types=[
            pltpu.VMEM((C,), jnp.int32),
            pltpu.VMEM((C,), jnp.int32),
            pltpu.VMEM((C, D), jnp.float32),
            pltpu.VMEM((rows_per_tile, D), jnp.float32),
            pltpu.VMEM_SHARED((n_pad, D), jnp.float32),
            pltpu.SemaphoreType.DMA,
        ],
        compiler_params=_SC_PARAMS,
    )
    def k(g_hbm, src_hbm, dst_hbm, out_hbm, src_v, dst_v, rows_v, zero_v,
          acc_sh, sem):
        cid = lax.axis_index("c")
        sid = lax.axis_index("s")
        wid = cid * NS + sid

        @pl.loop(0, rows_per_tile)
        def _(i):
            zero_v[i, :] = jnp.zeros((D,), jnp.float32)

        pltpu.sync_copy(zero_v, acc_sh.at[pl.ds(sid * rows_per_tile,
                                                rows_per_tile)])
        plsc.subcore_barrier()

        base = wid * chunks_per_w

        @pl.loop(0, chunks_per_w)
        def _(ci):
            off = (base + ci) * C
            pltpu.sync_copy(src_hbm.at[pl.ds(off, C)], src_v)
            pltpu.sync_copy(dst_hbm.at[pl.ds(off, C)], dst_v)
            pltpu.async_copy(g_hbm.at[src_v], rows_v, sem).wait()
            pltpu.sync_copy(rows_v, acc_sh.at[dst_v], add=True)

        plsc.subcore_barrier()
        pltpu.sync_copy(
            acc_sh.at[pl.ds(sid * rows_per_tile, rows_per_tile)],
            out_hbm.at[cid].at[pl.ds(sid * rows_per_tile, rows_per_tile)])

    return k(g, src, dst)


def _sc_degree(dst, n_pad, chunks_per_w):
    mesh = plsc.VectorSubcoreMesh(core_axis_name="c", subcore_axis_name="s")
    rows_per_tile = n_pad // NS

    @functools.partial(
        pl.kernel,
        out_type=jax.ShapeDtypeStruct((NC, n_pad, D), jnp.float32),
        mesh=mesh,
        scratch_types=[
            pltpu.VMEM((C,), jnp.int32),
            pltpu.VMEM((C, D), jnp.float32),
            pltpu.VMEM((rows_per_tile, D), jnp.float32),
            pltpu.VMEM_SHARED((n_pad, D), jnp.float32),
        ],
        compiler_params=_SC_PARAMS,
    )
    def k(dst_hbm, out_hbm, dst_v, ones_v, zero_v, acc_sh):
        cid = lax.axis_index("c")
        sid = lax.axis_index("s")
        wid = cid * NS + sid

        @pl.loop(0, rows_per_tile)
        def _(i):
            zero_v[i, :] = jnp.zeros((D,), jnp.float32)

        @pl.loop(0, C)
        def _(i):
            ones_v[i, :] = jnp.ones((D,), jnp.float32)

        pltpu.sync_copy(zero_v, acc_sh.at[pl.ds(sid * rows_per_tile,
                                                rows_per_tile)])
        plsc.subcore_barrier()

        base = wid * chunks_per_w

        @pl.loop(0, chunks_per_w)
        def _(ci):
            off = (base + ci) * C
            pltpu.sync_copy(dst_hbm.at[pl.ds(off, C)], dst_v)
            pltpu.sync_copy(ones_v, acc_sh.at[dst_v], add=True)

        plsc.subcore_barrier()
        pltpu.sync_copy(
            acc_sh.at[pl.ds(sid * rows_per_tile, rows_per_tile)],
            out_hbm.at[cid].at[pl.ds(sid * rows_per_tile, rows_per_tile)])

    return k(dst)


def _tc_matmul1(x, W1, n, n_pad):
    def body(x_ref, w_ref, o_ref):
        o_ref[0:n, :] = jnp.dot(x_ref[...], w_ref[...],
                                preferred_element_type=jnp.float32)
        o_ref[n:n_pad, :] = jnp.zeros((n_pad - n, D), jnp.float32)

    return pl.pallas_call(
        body,
        out_shape=jax.ShapeDtypeStruct((n_pad, D), jnp.float32),
    )(x, W1)


def _tc_norm(h1, degp, n, n_pad):
    def body(h_ref, d_ref, g_ref, dinv_ref):
        deg = d_ref[0] + d_ref[1] + 1.0
        dinv = lax.rsqrt(deg)
        dinv_ref[...] = dinv
        g_ref[...] = h_ref[...] * dinv

    return pl.pallas_call(
        body,
        out_shape=(jax.ShapeDtypeStruct((n_pad, D), jnp.float32),
                   jax.ShapeDtypeStruct((n_pad, D), jnp.float32)),
    )(h1, degp)


def _tc_mid(accp, g1, dinv, b1, W2, n, n_pad):
    def body(a_ref, g_ref, dinv_ref, b_ref, w_ref, o_ref):
        pre = dinv_ref[...] * (a_ref[0] + a_ref[1] + g_ref[...]) + b_ref[0:1, :]
        out1 = jnp.maximum(pre, 0.0)
        h2 = jnp.dot(out1, w_ref[...], preferred_element_type=jnp.float32)
        g2 = h2 * dinv_ref[...]
        o_ref[0:n, :] = g2[0:n, :]
        o_ref[n:n_pad, :] = jnp.zeros((n_pad - n, D), jnp.float32)

    return pl.pallas_call(
        body,
        out_shape=jax.ShapeDtypeStruct((n_pad, D), jnp.float32),
    )(accp, g1, dinv, b1, W2)


def _tc_final(accp, g2, dinv, b2, n):
    def body(a_ref, g_ref, dinv_ref, b_ref, o_ref):
        z = dinv_ref[...] * (a_ref[0] + a_ref[1] + g_ref[...]) + b_ref[0:1, :]
        z = z[0:n, :]
        m = jnp.max(z, axis=1, keepdims=True)
        s = z - m
        lse = jnp.log(jnp.sum(jnp.exp(s), axis=1, keepdims=True))
        o_ref[...] = s - lse

    return pl.pallas_call(
        body,
        out_shape=jax.ShapeDtypeStruct((n, D), jnp.float32),
    )(accp, g2, dinv, b2)


def kernel(x, edge_index, W1, b1, W2, b2):
    n = x.shape[0]
    e = edge_index.shape[1]
    n_pad = _pad_rows(n)

    e_pad = -(-e // (NW * C)) * (NW * C)
    chunks_per_w = e_pad // (NW * C)
    src = edge_index[0].astype(jnp.int32)
    dst = edge_index[1].astype(jnp.int32)
    if e_pad > e:
        pad_idx = (jnp.arange(e_pad - e, dtype=jnp.int32) % 8) + n
        src = jnp.concatenate([src, pad_idx])
        dst = jnp.concatenate([dst, pad_idx])

    b1r = b1.reshape(1, D)
    b2r = b2.reshape(1, D)

    degp = _sc_degree(dst, n_pad, chunks_per_w)
    h1 = _tc_matmul1(x, W1, n, n_pad)
    g1, dinv = _tc_norm(h1, degp, n, n_pad)
    acc1 = _sc_scatter_add(g1, src, dst, n_pad, chunks_per_w)
    g2 = _tc_mid(acc1, g1, dinv, b1r, W2, n, n_pad)
    acc2 = _sc_scatter_add(g2, src, dst, n_pad, chunks_per_w)
    return _tc_final(acc2, g2, dinv, b2r, n)
